# baseline (speedup 1.0000x reference)
import jax
import jax.numpy as jnp
import numpy as np
from jax import lax
from jax.experimental import pallas as pl
from jax.experimental.pallas import tpu as pltpu
from jax.experimental.pallas import tpu_sc as plsc

_RES = 256
_TUPLE = 16
_OBS_DIM = 8
_INPUT_DIM = _OBS_DIM * _RES
_N_NODES = _INPUT_DIM // _TUPLE
_NUM_OUT = 8
_ENC_MIN, _ENC_MAX = -1.0, 1.0


def _keys_body(obs_ref, onehot_ref, thr_ref, w_ref, keys_ref):
    vals = jnp.dot(obs_ref[...], onehot_ref[...],
                   precision=lax.Precision.HIGHEST,
                   preferred_element_type=jnp.float32)
    bits = (vals > thr_ref[...]).astype(jnp.bfloat16)
    keys_f = jnp.dot(bits, w_ref[...], preferred_element_type=jnp.float32)
    offs = lax.broadcasted_iota(jnp.int32, keys_f.shape, 1) * (2 ** _TUPLE)
    keys_ref[...] = keys_f.astype(jnp.int32) + offs


def _compute_keys(obs, mapping):
    B = obs.shape[0]
    perm_d = mapping // _RES
    perm_t = mapping % _RES
    thr = _ENC_MIN + (_ENC_MAX - _ENC_MIN) * (
        jnp.arange(_RES, dtype=jnp.float32) + 1.0) / (_RES + 1.0)
    thr_perm = thr[perm_t].reshape(1, _INPUT_DIM)
    onehot = (perm_d[None, :] == jnp.arange(_OBS_DIM, dtype=jnp.int32)[:, None]
              ).astype(jnp.float32)
    j = np.arange(_INPUT_DIM)
    w_np = np.where((j[:, None] // _TUPLE) == np.arange(_N_NODES)[None, :],
                    (2.0 ** (j % _TUPLE))[:, None], 0.0)
    w = jnp.asarray(w_np, dtype=jnp.bfloat16)

    bt = 512
    return pl.pallas_call(
        _keys_body,
        grid=(B // bt,),
        in_specs=[
            pl.BlockSpec((bt, _OBS_DIM), lambda i: (i, 0)),
            pl.BlockSpec((_OBS_DIM, _INPUT_DIM), lambda i: (0, 0)),
            pl.BlockSpec((1, _INPUT_DIM), lambda i: (0, 0)),
            pl.BlockSpec((_INPUT_DIM, _N_NODES), lambda i: (0, 0)),
        ],
        out_specs=pl.BlockSpec((bt, _N_NODES), lambda i: (i, 0)),
        out_shape=jax.ShapeDtypeStruct((B, _N_NODES), jnp.int32),
    )(obs, onehot, thr_perm, w)


_CH = 16
_NW = 32
_GRP = _N_NODES // 16


_KPC = _CH * _N_NODES


def _bag_body(keys_hbm, pol_hbm, vf_hbm, pol_out, vf_out, *scr):
    pidx = scr[:_NUM_OUT]
    gath = scr[_NUM_OUT:2 * _NUM_OUT]
    keys_v, vfg, polbuf, vfbuf, sem = scr[2 * _NUM_OUT:]
    nc = 2
    wid = lax.axis_index("s") * nc + lax.axis_index("c")
    spw = keys_hbm.shape[0] // (_NW * _N_NODES)
    nchunk = spw // _CH

    def chunk_body(ck, carry):
        s0 = wid * spw + ck * _CH
        pltpu.sync_copy(keys_hbm.at[pl.ds(s0 * _N_NODES, _KPC)], keys_v)

        def idx_body(j, c2):
            kk = keys_v[pl.ds(j * 16, 16)]
            b0 = ((kk >> 7) << 10) | (kk & 127)
            for comp in range(_NUM_OUT):
                pidx[comp][pl.ds(j * 16, 16)] = b0 + (comp * 128)
            return c2
        lax.fori_loop(0, _KPC // 16, idx_body, 0)

        for comp in range(_NUM_OUT):
            pltpu.async_copy(pol_hbm.at[pidx[comp]], gath[comp], sem)
        pltpu.async_copy(vf_hbm.at[keys_v], vfg, sem)
        for comp in range(_NUM_OUT):
            pltpu.make_async_copy(pol_hbm.at[pidx[comp]],
                                  gath[comp], sem).wait()
        pltpu.make_async_copy(vf_hbm.at[keys_v], vfg, sem).wait()

        lane = lax.iota(jnp.int32, 16)
        samp_base = lane * _N_NODES

        def acc_body(n, accs):
            kvec = samp_base + n
            pol_accs = tuple(
                accs[comp] + plsc.load_gather(gath[comp], [kvec])
                for comp in range(_NUM_OUT))
            return pol_accs + (accs[_NUM_OUT] + plsc.load_gather(vfg, [kvec]),)

        zero = jnp.zeros((16,), jnp.float32)
        accs = lax.fori_loop(0, _N_NODES, acc_body, (zero,) * (_NUM_OUT + 1))
        for comp in range(_NUM_OUT):
            plsc.store_scatter(polbuf, [lane * _NUM_OUT + comp], accs[comp])
        vfbuf[...] = accs[_NUM_OUT]

        pltpu.sync_copy(polbuf, pol_out.at[pl.ds(s0 * _NUM_OUT,
                                                 _CH * _NUM_OUT)])
        pltpu.sync_copy(vfbuf, vf_out.at[pl.ds(s0, _CH)])
        return carry
    lax.fori_loop(0, nchunk, chunk_body, 0)


def _bag(keys, pol_flat, vf_flat):
    B = keys.shape[0]
    mesh = plsc.VectorSubcoreMesh(core_axis_name="c", subcore_axis_name="s")
    f = pl.kernel(
        _bag_body,
        out_type=(jax.ShapeDtypeStruct((B * _NUM_OUT,), jnp.float32),
                  jax.ShapeDtypeStruct((B,), jnp.float32)),
        mesh=mesh,
        compiler_params=pltpu.CompilerParams(needs_layout_passes=False),
        scratch_types=(
            [pltpu.VMEM((_KPC,), jnp.int32)] * _NUM_OUT +
            [pltpu.VMEM((_KPC,), jnp.float32)] * _NUM_OUT +
            [
                pltpu.VMEM((_KPC,), jnp.int32),
                pltpu.VMEM((_KPC,), jnp.float32),
                pltpu.VMEM((_CH * _NUM_OUT,), jnp.float32),
                pltpu.VMEM((_CH,), jnp.float32),
                pltpu.SemaphoreType.DMA,
            ]),
    )
    return f(keys.reshape(-1), pol_flat, vf_flat)


def kernel(obs, policy_weight, vf_weight, mapping):
    B = obs.shape[0]
    keys = _compute_keys(obs, mapping)
    n_emb = policy_weight.shape[0]
    pol_flat = (policy_weight.reshape(n_emb // 128, 128, _NUM_OUT)
                .transpose(0, 2, 1).reshape(-1))
    vf_flat = vf_weight.reshape(-1)
    pol_out, vf_out = _bag(keys, pol_flat, vf_flat)
    return pol_out.reshape(B, _NUM_OUT), vf_out


# speedup vs baseline: 1.2077x; 1.2077x over previous
import jax
import jax.numpy as jnp
import numpy as np
from jax import lax
from jax.experimental import pallas as pl
from jax.experimental.pallas import tpu as pltpu
from jax.experimental.pallas import tpu_sc as plsc

_RES = 256
_TUPLE = 16
_OBS_DIM = 8
_INPUT_DIM = _OBS_DIM * _RES
_N_NODES = _INPUT_DIM // _TUPLE
_NUM_OUT = 8
_ENC_MIN, _ENC_MAX = -1.0, 1.0


def _keys_body(obs_ref, onehot_ref, thr_ref, w_ref, keys_ref):
    vals = jnp.dot(obs_ref[...], onehot_ref[...],
                   precision=lax.Precision.HIGHEST,
                   preferred_element_type=jnp.float32)
    bits = (vals > thr_ref[...]).astype(jnp.bfloat16)
    keys_f = jnp.dot(bits, w_ref[...], preferred_element_type=jnp.float32)
    offs = lax.broadcasted_iota(jnp.int32, keys_f.shape, 1) * (2 ** _TUPLE)
    keys_ref[...] = keys_f.astype(jnp.int32) + offs


def _compute_keys(obs, mapping):
    B = obs.shape[0]
    perm_d = mapping // _RES
    perm_t = mapping % _RES
    thr = _ENC_MIN + (_ENC_MAX - _ENC_MIN) * (
        jnp.arange(_RES, dtype=jnp.float32) + 1.0) / (_RES + 1.0)
    thr_perm = thr[perm_t].reshape(1, _INPUT_DIM)
    onehot = (perm_d[None, :] == jnp.arange(_OBS_DIM, dtype=jnp.int32)[:, None]
              ).astype(jnp.float32)
    j = np.arange(_INPUT_DIM)
    w_np = np.where((j[:, None] // _TUPLE) == np.arange(_N_NODES)[None, :],
                    (2.0 ** (j % _TUPLE))[:, None], 0.0)
    w = jnp.asarray(w_np, dtype=jnp.bfloat16)

    bt = 512
    return pl.pallas_call(
        _keys_body,
        grid=(B // bt,),
        in_specs=[
            pl.BlockSpec((bt, _OBS_DIM), lambda i: (i, 0)),
            pl.BlockSpec((_OBS_DIM, _INPUT_DIM), lambda i: (0, 0)),
            pl.BlockSpec((1, _INPUT_DIM), lambda i: (0, 0)),
            pl.BlockSpec((_INPUT_DIM, _N_NODES), lambda i: (0, 0)),
        ],
        out_specs=pl.BlockSpec((bt, _N_NODES), lambda i: (i, 0)),
        out_shape=jax.ShapeDtypeStruct((B, _N_NODES), jnp.int32),
    )(obs, onehot, thr_perm, w)


_CH = 16
_NW = 32
_GRP = _N_NODES // 16


_KPC = _CH * _N_NODES


def _bag_body(keys_hbm, pol_hbm, vf_hbm, pol_out, vf_out, *scr):
    nbuf = _NUM_OUT + _NUM_OUT + 2
    bufs = [scr[b * nbuf:(b + 1) * nbuf] for b in range(2)]
    polbuf, vfbuf = scr[2 * nbuf:2 * nbuf + 2]
    sems = scr[2 * nbuf + 2:]
    nc = 2
    wid = lax.axis_index("s") * nc + lax.axis_index("c")
    spw = keys_hbm.shape[0] // (_NW * _N_NODES)
    nchunk = spw // _CH
    lane = lax.iota(jnp.int32, 16)
    zero = jnp.zeros((16,), jnp.float32)

    def prep_fire(ck, buf, sem):
        pidx = buf[:_NUM_OUT]
        gath = buf[_NUM_OUT:2 * _NUM_OUT]
        keys_v, vfg = buf[2 * _NUM_OUT:]
        g0 = (wid * nchunk + ck) * _KPC
        pltpu.sync_copy(keys_hbm.at[pl.ds(g0, _KPC)], keys_v)

        def idx_body(j, c2):
            kk = keys_v[pl.ds(j * 16, 16)]
            b0 = ((kk >> 7) << 10) | (kk & 127)
            for comp in range(_NUM_OUT):
                pidx[comp][pl.ds(j * 16, 16)] = b0 + (comp * 128)
            return c2
        lax.fori_loop(0, _KPC // 16, idx_body, 0)

        for comp in range(_NUM_OUT):
            pltpu.async_copy(pol_hbm.at[pidx[comp]], gath[comp], sem)
        pltpu.async_copy(vf_hbm.at[keys_v], vfg, sem)

    def finish(ck, buf, sem):
        pidx = buf[:_NUM_OUT]
        gath = buf[_NUM_OUT:2 * _NUM_OUT]
        keys_v, vfg = buf[2 * _NUM_OUT:]
        for comp in range(_NUM_OUT):
            pltpu.make_async_copy(pol_hbm.at[pidx[comp]],
                                  gath[comp], sem).wait()
        pltpu.make_async_copy(vf_hbm.at[keys_v], vfg, sem).wait()

        def acc_body(j, accs):
            pol_accs = tuple(
                accs[comp] + gath[comp][pl.ds(j * 16, 16)]
                for comp in range(_NUM_OUT))
            return pol_accs + (accs[_NUM_OUT] + vfg[pl.ds(j * 16, 16)],)
        accs = lax.fori_loop(0, _N_NODES, acc_body, (zero,) * (_NUM_OUT + 1))
        for comp in range(_NUM_OUT):
            plsc.store_scatter(polbuf, [lane * _NUM_OUT + comp], accs[comp])
        vfbuf[...] = accs[_NUM_OUT]

        s0 = wid * spw + ck * _CH
        pltpu.sync_copy(polbuf, pol_out.at[pl.ds(s0 * _NUM_OUT,
                                                 _CH * _NUM_OUT)])
        pltpu.sync_copy(vfbuf, vf_out.at[pl.ds(s0, _CH)])

    prep_fire(0, bufs[0], sems[0])

    def pair_body(ck2, carry):
        ck = ck2 * 2
        prep_fire(ck + 1, bufs[1], sems[1])
        finish(ck, bufs[0], sems[0])

        @pl.when(ck2 * 2 + 2 < nchunk)
        def _():
            prep_fire(ck + 2, bufs[0], sems[0])
        finish(ck + 1, bufs[1], sems[1])
        return carry
    lax.fori_loop(0, nchunk // 2, pair_body, 0)


def _bag(keys, pol_flat, vf_flat):
    B = keys.shape[0]
    mesh = plsc.VectorSubcoreMesh(core_axis_name="c", subcore_axis_name="s")
    f = pl.kernel(
        _bag_body,
        out_type=(jax.ShapeDtypeStruct((B * _NUM_OUT,), jnp.float32),
                  jax.ShapeDtypeStruct((B,), jnp.float32)),
        mesh=mesh,
        compiler_params=pltpu.CompilerParams(needs_layout_passes=False),
        scratch_types=(
            ([pltpu.VMEM((_KPC,), jnp.int32)] * _NUM_OUT +
             [pltpu.VMEM((_KPC,), jnp.float32)] * _NUM_OUT +
             [pltpu.VMEM((_KPC,), jnp.int32),
              pltpu.VMEM((_KPC,), jnp.float32)]
             ) * 2 +
            [
                pltpu.VMEM((_CH * _NUM_OUT,), jnp.float32),
                pltpu.VMEM((_CH,), jnp.float32),
                pltpu.SemaphoreType.DMA,
                pltpu.SemaphoreType.DMA,
            ]),
    )
    keys_t = (keys.reshape(B // _CH, _CH, _N_NODES)
              .transpose(0, 2, 1).reshape(-1))
    return f(keys_t, pol_flat, vf_flat)


def kernel(obs, policy_weight, vf_weight, mapping):
    B = obs.shape[0]
    keys = _compute_keys(obs, mapping)
    n_emb = policy_weight.shape[0]
    pol_flat = (policy_weight.reshape(n_emb // 128, 128, _NUM_OUT)
                .transpose(0, 2, 1).reshape(-1))
    vf_flat = vf_weight.reshape(-1)
    pol_out, vf_out = _bag(keys, pol_flat, vf_flat)
    return pol_out.reshape(B, _NUM_OUT), vf_out


# speedup vs baseline: 1.2183x; 1.0088x over previous
import jax
import jax.numpy as jnp
import numpy as np
from jax import lax
from jax.experimental import pallas as pl
from jax.experimental.pallas import tpu as pltpu
from jax.experimental.pallas import tpu_sc as plsc

_RES = 256
_TUPLE = 16
_OBS_DIM = 8
_INPUT_DIM = _OBS_DIM * _RES
_N_NODES = _INPUT_DIM // _TUPLE
_NUM_OUT = 8
_ENC_MIN, _ENC_MAX = -1.0, 1.0


def _keys_body(obs_ref, onehot_ref, thr_ref, pow_ref, keys_ref):
    vals = jnp.dot(obs_ref[...], onehot_ref[...],
                   precision=lax.Precision.HIGHEST,
                   preferred_element_type=jnp.float32)
    sel = jnp.where(vals > thr_ref[...], pow_ref[...], 0.0)
    acc = sel[:, 0:_N_NODES]
    for t in range(1, _TUPLE):
        acc = acc + sel[:, t * _N_NODES:(t + 1) * _N_NODES]
    offs = lax.broadcasted_iota(jnp.int32, acc.shape, 1) * (2 ** _TUPLE)
    keys_ref[...] = acc.astype(jnp.int32) + offs


def _compute_keys(obs, mapping):
    B = obs.shape[0]
    j2 = np.arange(_INPUT_DIM)
    order = jnp.asarray(((j2 % _N_NODES) * _TUPLE + j2 // _N_NODES)
                        .astype(np.int32))
    mapping_t = mapping[order]
    perm_d = mapping_t // _RES
    perm_t = mapping_t % _RES
    thr = _ENC_MIN + (_ENC_MAX - _ENC_MIN) * (
        jnp.arange(_RES, dtype=jnp.float32) + 1.0) / (_RES + 1.0)
    thr_perm = thr[perm_t].reshape(1, _INPUT_DIM)
    onehot = (perm_d[None, :] == jnp.arange(_OBS_DIM, dtype=jnp.int32)[:, None]
              ).astype(jnp.float32)
    pow_row = jnp.asarray((2.0 ** (j2 // _N_NODES))[None, :],
                          dtype=jnp.float32)

    bt = 512
    return pl.pallas_call(
        _keys_body,
        grid=(B // bt,),
        in_specs=[
            pl.BlockSpec((bt, _OBS_DIM), lambda i: (i, 0)),
            pl.BlockSpec((_OBS_DIM, _INPUT_DIM), lambda i: (0, 0)),
            pl.BlockSpec((1, _INPUT_DIM), lambda i: (0, 0)),
            pl.BlockSpec((1, _INPUT_DIM), lambda i: (0, 0)),
        ],
        out_specs=pl.BlockSpec((bt, _N_NODES), lambda i: (i, 0)),
        out_shape=jax.ShapeDtypeStruct((B, _N_NODES), jnp.int32),
    )(obs, onehot, thr_perm, pow_row)


_CH = 16
_NW = 32
_GRP = _N_NODES // 16


_KPC = _CH * _N_NODES


def _bag_body(keys_hbm, pol_hbm, vf_hbm, pol_out, vf_out, *scr):
    nbuf = _NUM_OUT + _NUM_OUT + 2
    bufs = [scr[b * nbuf:(b + 1) * nbuf] for b in range(2)]
    polbuf, vfbuf = scr[2 * nbuf:2 * nbuf + 2]
    sems = scr[2 * nbuf + 2:]
    nc = 2
    wid = lax.axis_index("s") * nc + lax.axis_index("c")
    spw = keys_hbm.shape[0] // (_NW * _N_NODES)
    nchunk = spw // _CH
    lane = lax.iota(jnp.int32, 16)
    zero = jnp.zeros((16,), jnp.float32)

    def prep_fire(ck, buf, sem):
        pidx = buf[:_NUM_OUT]
        gath = buf[_NUM_OUT:2 * _NUM_OUT]
        keys_v, vfg = buf[2 * _NUM_OUT:]
        g0 = (wid * nchunk + ck) * _KPC
        pltpu.sync_copy(keys_hbm.at[pl.ds(g0, _KPC)], keys_v)

        def idx_body(j, c2):
            kk = keys_v[pl.ds(j * 16, 16)]
            b0 = ((kk >> 7) << 10) | (kk & 127)
            for comp in range(_NUM_OUT):
                pidx[comp][pl.ds(j * 16, 16)] = b0 + (comp * 128)
            return c2
        lax.fori_loop(0, _KPC // 16, idx_body, 0)

        for comp in range(_NUM_OUT):
            pltpu.async_copy(pol_hbm.at[pidx[comp]], gath[comp], sem)
        pltpu.async_copy(vf_hbm.at[keys_v], vfg, sem)

    def finish(ck, buf, sem):
        pidx = buf[:_NUM_OUT]
        gath = buf[_NUM_OUT:2 * _NUM_OUT]
        keys_v, vfg = buf[2 * _NUM_OUT:]
        for comp in range(_NUM_OUT):
            pltpu.make_async_copy(pol_hbm.at[pidx[comp]],
                                  gath[comp], sem).wait()
        pltpu.make_async_copy(vf_hbm.at[keys_v], vfg, sem).wait()

        def acc_body(j, accs):
            pol_accs = tuple(
                accs[comp] + gath[comp][pl.ds(j * 16, 16)]
                for comp in range(_NUM_OUT))
            return pol_accs + (accs[_NUM_OUT] + vfg[pl.ds(j * 16, 16)],)
        accs = lax.fori_loop(0, _N_NODES, acc_body, (zero,) * (_NUM_OUT + 1))
        for comp in range(_NUM_OUT):
            plsc.store_scatter(polbuf, [lane * _NUM_OUT + comp], accs[comp])
        vfbuf[...] = accs[_NUM_OUT]

        s0 = wid * spw + ck * _CH
        pltpu.sync_copy(polbuf, pol_out.at[pl.ds(s0 * _NUM_OUT,
                                                 _CH * _NUM_OUT)])
        pltpu.sync_copy(vfbuf, vf_out.at[pl.ds(s0, _CH)])

    prep_fire(0, bufs[0], sems[0])

    def pair_body(ck2, carry):
        ck = ck2 * 2
        prep_fire(ck + 1, bufs[1], sems[1])
        finish(ck, bufs[0], sems[0])

        @pl.when(ck2 * 2 + 2 < nchunk)
        def _():
            prep_fire(ck + 2, bufs[0], sems[0])
        finish(ck + 1, bufs[1], sems[1])
        return carry
    lax.fori_loop(0, nchunk // 2, pair_body, 0)


def _bag(keys, pol_flat, vf_flat):
    B = keys.shape[0]
    mesh = plsc.VectorSubcoreMesh(core_axis_name="c", subcore_axis_name="s")
    f = pl.kernel(
        _bag_body,
        out_type=(jax.ShapeDtypeStruct((B * _NUM_OUT,), jnp.float32),
                  jax.ShapeDtypeStruct((B,), jnp.float32)),
        mesh=mesh,
        compiler_params=pltpu.CompilerParams(needs_layout_passes=False),
        scratch_types=(
            ([pltpu.VMEM((_KPC,), jnp.int32)] * _NUM_OUT +
             [pltpu.VMEM((_KPC,), jnp.float32)] * _NUM_OUT +
             [pltpu.VMEM((_KPC,), jnp.int32),
              pltpu.VMEM((_KPC,), jnp.float32)]
             ) * 2 +
            [
                pltpu.VMEM((_CH * _NUM_OUT,), jnp.float32),
                pltpu.VMEM((_CH,), jnp.float32),
                pltpu.SemaphoreType.DMA,
                pltpu.SemaphoreType.DMA,
            ]),
    )
    keys_t = (keys.reshape(B // _CH, _CH, _N_NODES)
              .transpose(0, 2, 1).reshape(-1))
    return f(keys_t, pol_flat, vf_flat)


def kernel(obs, policy_weight, vf_weight, mapping):
    B = obs.shape[0]
    keys = _compute_keys(obs, mapping)
    n_emb = policy_weight.shape[0]
    pol_flat = (policy_weight.reshape(n_emb // 128, 128, _NUM_OUT)
                .transpose(0, 2, 1).reshape(-1))
    vf_flat = vf_weight.reshape(-1)
    pol_out, vf_out = _bag(keys, pol_flat, vf_flat)
    return pol_out.reshape(B, _NUM_OUT), vf_out


# speedup vs baseline: 1.2644x; 1.0378x over previous
import jax
import jax.numpy as jnp
import numpy as np
from jax import lax
from jax.experimental import pallas as pl
from jax.experimental.pallas import tpu as pltpu
from jax.experimental.pallas import tpu_sc as plsc

_RES = 256
_TUPLE = 16
_OBS_DIM = 8
_INPUT_DIM = _OBS_DIM * _RES
_N_NODES = _INPUT_DIM // _TUPLE
_NUM_OUT = 8
_ENC_MIN, _ENC_MAX = -1.0, 1.0


def _keys_body(obs_ref, onehot_ref, thr_ref, pow_ref, keys_ref):
    vals = jnp.dot(obs_ref[...], onehot_ref[...],
                   precision=lax.Precision.HIGHEST,
                   preferred_element_type=jnp.float32)
    sel = jnp.where(vals > thr_ref[...], pow_ref[...], 0.0)
    acc = sel[:, 0:_N_NODES]
    for t in range(1, _TUPLE):
        acc = acc + sel[:, t * _N_NODES:(t + 1) * _N_NODES]
    offs = lax.broadcasted_iota(jnp.int32, acc.shape, 1) * (2 ** _TUPLE)
    keys_ref[...] = acc.astype(jnp.int32) + offs


def _compute_keys(obs, mapping, row0, rows):
    B = obs.shape[0]
    j2 = np.arange(_INPUT_DIM)
    order = jnp.asarray(((j2 % _N_NODES) * _TUPLE + j2 // _N_NODES)
                        .astype(np.int32))
    mapping_t = mapping[order]
    perm_d = mapping_t // _RES
    perm_t = mapping_t % _RES
    thr = _ENC_MIN + (_ENC_MAX - _ENC_MIN) * (
        jnp.arange(_RES, dtype=jnp.float32) + 1.0) / (_RES + 1.0)
    thr_perm = thr[perm_t].reshape(1, _INPUT_DIM)
    onehot = (perm_d[None, :] == jnp.arange(_OBS_DIM, dtype=jnp.int32)[:, None]
              ).astype(jnp.float32)
    pow_row = jnp.asarray((2.0 ** (j2 // _N_NODES))[None, :],
                          dtype=jnp.float32)

    bt = 512
    base = row0 // bt
    return pl.pallas_call(
        _keys_body,
        grid=(rows // bt,),
        in_specs=[
            pl.BlockSpec((bt, _OBS_DIM), lambda i: (i + base, 0)),
            pl.BlockSpec((_OBS_DIM, _INPUT_DIM), lambda i: (0, 0)),
            pl.BlockSpec((1, _INPUT_DIM), lambda i: (0, 0)),
            pl.BlockSpec((1, _INPUT_DIM), lambda i: (0, 0)),
        ],
        out_specs=pl.BlockSpec((bt, _N_NODES), lambda i: (i, 0)),
        out_shape=jax.ShapeDtypeStruct((rows, _N_NODES), jnp.int32),
    )(obs, onehot, thr_perm, pow_row)


_CH = 16
_NW = 32
_GRP = _N_NODES // 16


_KPC = _CH * _N_NODES


def _bag_body(keys_hbm, pol_hbm, vf_hbm, pol_out, vf_out, *scr):
    nbuf = _NUM_OUT + _NUM_OUT + 2
    bufs = [scr[b * nbuf:(b + 1) * nbuf] for b in range(2)]
    polbuf, vfbuf = scr[2 * nbuf:2 * nbuf + 2]
    sems = scr[2 * nbuf + 2:]
    nc = 2
    wid = lax.axis_index("s") * nc + lax.axis_index("c")
    spw = keys_hbm.shape[0] // (_NW * _N_NODES)
    nchunk = spw // _CH
    lane = lax.iota(jnp.int32, 16)
    zero = jnp.zeros((16,), jnp.float32)

    def prep_fire(ck, buf, sem):
        pidx = buf[:_NUM_OUT]
        gath = buf[_NUM_OUT:2 * _NUM_OUT]
        keys_v, vfg = buf[2 * _NUM_OUT:]
        g0 = (wid * nchunk + ck) * _KPC
        pltpu.sync_copy(keys_hbm.at[pl.ds(g0, _KPC)], keys_v)

        def idx_body(j, c2):
            kk = keys_v[pl.ds(j * 16, 16)]
            b0 = ((kk >> 7) << 10) | (kk & 127)
            for comp in range(_NUM_OUT):
                pidx[comp][pl.ds(j * 16, 16)] = b0 + (comp * 128)
            return c2
        lax.fori_loop(0, _KPC // 16, idx_body, 0)

        for comp in range(_NUM_OUT):
            pltpu.async_copy(pol_hbm.at[pidx[comp]], gath[comp], sem)
        pltpu.async_copy(vf_hbm.at[keys_v], vfg, sem)

    def finish(ck, buf, sem):
        pidx = buf[:_NUM_OUT]
        gath = buf[_NUM_OUT:2 * _NUM_OUT]
        keys_v, vfg = buf[2 * _NUM_OUT:]
        for comp in range(_NUM_OUT):
            pltpu.make_async_copy(pol_hbm.at[pidx[comp]],
                                  gath[comp], sem).wait()
        pltpu.make_async_copy(vf_hbm.at[keys_v], vfg, sem).wait()

        def acc_body(j, accs):
            pol_accs = tuple(
                accs[comp] + gath[comp][pl.ds(j * 16, 16)]
                for comp in range(_NUM_OUT))
            return pol_accs + (accs[_NUM_OUT] + vfg[pl.ds(j * 16, 16)],)
        accs = lax.fori_loop(0, _N_NODES, acc_body, (zero,) * (_NUM_OUT + 1))
        for comp in range(_NUM_OUT):
            plsc.store_scatter(polbuf, [lane * _NUM_OUT + comp], accs[comp])
        vfbuf[...] = accs[_NUM_OUT]

        s0 = wid * spw + ck * _CH
        pltpu.sync_copy(polbuf, pol_out.at[pl.ds(s0 * _NUM_OUT,
                                                 _CH * _NUM_OUT)])
        pltpu.sync_copy(vfbuf, vf_out.at[pl.ds(s0, _CH)])

    prep_fire(0, bufs[0], sems[0])

    def pair_body(ck2, carry):
        ck = ck2 * 2
        prep_fire(ck + 1, bufs[1], sems[1])
        finish(ck, bufs[0], sems[0])

        @pl.when(ck2 * 2 + 2 < nchunk)
        def _():
            prep_fire(ck + 2, bufs[0], sems[0])
        finish(ck + 1, bufs[1], sems[1])
        return carry
    lax.fori_loop(0, nchunk // 2, pair_body, 0)


def _bag(keys, pol_flat, vf_flat):
    B = keys.shape[0]
    mesh = plsc.VectorSubcoreMesh(core_axis_name="c", subcore_axis_name="s")
    f = pl.kernel(
        _bag_body,
        out_type=(jax.ShapeDtypeStruct((B * _NUM_OUT,), jnp.float32),
                  jax.ShapeDtypeStruct((B,), jnp.float32)),
        mesh=mesh,
        compiler_params=pltpu.CompilerParams(needs_layout_passes=False),
        scratch_types=(
            ([pltpu.VMEM((_KPC,), jnp.int32)] * _NUM_OUT +
             [pltpu.VMEM((_KPC,), jnp.float32)] * _NUM_OUT +
             [pltpu.VMEM((_KPC,), jnp.int32),
              pltpu.VMEM((_KPC,), jnp.float32)]
             ) * 2 +
            [
                pltpu.VMEM((_CH * _NUM_OUT,), jnp.float32),
                pltpu.VMEM((_CH,), jnp.float32),
                pltpu.SemaphoreType.DMA,
                pltpu.SemaphoreType.DMA,
            ]),
    )
    keys_t = (keys.reshape(B // _CH, _CH, _N_NODES)
              .transpose(0, 2, 1).reshape(-1))
    return f(keys_t, pol_flat, vf_flat)


def kernel(obs, policy_weight, vf_weight, mapping):
    B = obs.shape[0]
    n_emb = policy_weight.shape[0]
    pol_flat = (policy_weight.reshape(n_emb // 128, 128, _NUM_OUT)
                .transpose(0, 2, 1).reshape(-1))
    vf_flat = vf_weight.reshape(-1)
    h = B // 2
    pol_parts, vf_parts = [], []
    for part in range(2):
        keys = _compute_keys(obs, mapping, part * h, h)
        pol_p, vf_p = _bag(keys, pol_flat, vf_flat)
        pol_parts.append(pol_p)
        vf_parts.append(vf_p)
    pol_out = jnp.concatenate(pol_parts).reshape(B, _NUM_OUT)
    vf_out = jnp.concatenate(vf_parts)
    return pol_out, vf_out


# speedup vs baseline: 1.2874x; 1.0182x over previous
import jax
import jax.numpy as jnp
import numpy as np
from jax import lax
from jax.experimental import pallas as pl
from jax.experimental.pallas import tpu as pltpu
from jax.experimental.pallas import tpu_sc as plsc

_RES = 256
_TUPLE = 16
_OBS_DIM = 8
_INPUT_DIM = _OBS_DIM * _RES
_N_NODES = _INPUT_DIM // _TUPLE
_NUM_OUT = 8
_ENC_MIN, _ENC_MAX = -1.0, 1.0


def _keys_body(obs_ref, onehot_ref, thr_ref, pow_ref, keys_ref):
    vals = jnp.dot(obs_ref[...], onehot_ref[...],
                   precision=lax.Precision.HIGHEST,
                   preferred_element_type=jnp.float32)
    sel = jnp.where(vals > thr_ref[...], pow_ref[...], 0.0)
    acc = sel[:, 0:_N_NODES]
    for t in range(1, _TUPLE):
        acc = acc + sel[:, t * _N_NODES:(t + 1) * _N_NODES]
    offs = lax.broadcasted_iota(jnp.int32, acc.shape, 1) * (2 ** _TUPLE)
    keys_ref[...] = acc.astype(jnp.int32) + offs


def _compute_keys(obs, mapping, row0, rows):
    B = obs.shape[0]
    j2 = np.arange(_INPUT_DIM)
    order = jnp.asarray(((j2 % _N_NODES) * _TUPLE + j2 // _N_NODES)
                        .astype(np.int32))
    mapping_t = mapping[order]
    perm_d = mapping_t // _RES
    perm_t = mapping_t % _RES
    thr = _ENC_MIN + (_ENC_MAX - _ENC_MIN) * (
        jnp.arange(_RES, dtype=jnp.float32) + 1.0) / (_RES + 1.0)
    thr_perm = thr[perm_t].reshape(1, _INPUT_DIM)
    onehot = (perm_d[None, :] == jnp.arange(_OBS_DIM, dtype=jnp.int32)[:, None]
              ).astype(jnp.float32)
    pow_row = jnp.asarray((2.0 ** (j2 // _N_NODES))[None, :],
                          dtype=jnp.float32)

    bt = 512
    base = row0 // bt
    return pl.pallas_call(
        _keys_body,
        grid=(rows // bt,),
        in_specs=[
            pl.BlockSpec((bt, _OBS_DIM), lambda i: (i + base, 0)),
            pl.BlockSpec((_OBS_DIM, _INPUT_DIM), lambda i: (0, 0)),
            pl.BlockSpec((1, _INPUT_DIM), lambda i: (0, 0)),
            pl.BlockSpec((1, _INPUT_DIM), lambda i: (0, 0)),
        ],
        out_specs=pl.BlockSpec((bt, _N_NODES), lambda i: (i, 0)),
        out_shape=jax.ShapeDtypeStruct((rows, _N_NODES), jnp.int32),
    )(obs, onehot, thr_perm, pow_row)


_CH = 16
_NW = 32
_GRP = _N_NODES // 16


_KPC = _CH * _N_NODES


def _bag_body(keys_hbm, pol_hbm, vf_hbm, pol_out, vf_out, *scr):
    nbuf = _NUM_OUT + _NUM_OUT + 2
    bufs = [scr[b * nbuf:(b + 1) * nbuf] for b in range(2)]
    polbuf, vfbuf = scr[2 * nbuf:2 * nbuf + 2]
    sems = scr[2 * nbuf + 2:]
    nc = 2
    wid = lax.axis_index("s") * nc + lax.axis_index("c")
    spw = keys_hbm.shape[0] // (_NW * _N_NODES)
    nchunk = spw // _CH
    lane = lax.iota(jnp.int32, 16)
    zero = jnp.zeros((16,), jnp.float32)

    def prep_fire(ck, buf, sem):
        pidx = buf[:_NUM_OUT]
        gath = buf[_NUM_OUT:2 * _NUM_OUT]
        keys_v, vfg = buf[2 * _NUM_OUT:]
        g0 = (wid * nchunk + ck) * _KPC
        pltpu.sync_copy(keys_hbm.at[pl.ds(g0, _KPC)], keys_v)

        def idx_body(j, c2):
            kk = keys_v[pl.ds(j * 16, 16)]
            b0 = ((kk >> 7) << 10) | (kk & 127)
            for comp in range(_NUM_OUT):
                pidx[comp][pl.ds(j * 16, 16)] = b0 + (comp * 128)
            return c2
        lax.fori_loop(0, _KPC // 16, idx_body, 0)

        for comp in range(_NUM_OUT):
            pltpu.async_copy(pol_hbm.at[pidx[comp]], gath[comp], sem)
        pltpu.async_copy(vf_hbm.at[keys_v], vfg, sem)

    def finish(ck, buf, sem):
        pidx = buf[:_NUM_OUT]
        gath = buf[_NUM_OUT:2 * _NUM_OUT]
        keys_v, vfg = buf[2 * _NUM_OUT:]
        for comp in range(_NUM_OUT):
            pltpu.make_async_copy(pol_hbm.at[pidx[comp]],
                                  gath[comp], sem).wait()
        pltpu.make_async_copy(vf_hbm.at[keys_v], vfg, sem).wait()

        def acc_body(j, accs):
            pol_accs = tuple(
                accs[comp] + gath[comp][pl.ds(j * 16, 16)]
                for comp in range(_NUM_OUT))
            return pol_accs + (accs[_NUM_OUT] + vfg[pl.ds(j * 16, 16)],)
        accs = lax.fori_loop(0, _N_NODES, acc_body, (zero,) * (_NUM_OUT + 1))
        for comp in range(_NUM_OUT):
            plsc.store_scatter(polbuf, [lane * _NUM_OUT + comp], accs[comp])
        vfbuf[...] = accs[_NUM_OUT]

        s0 = wid * spw + ck * _CH
        pltpu.sync_copy(polbuf, pol_out.at[pl.ds(s0 * _NUM_OUT,
                                                 _CH * _NUM_OUT)])
        pltpu.sync_copy(vfbuf, vf_out.at[pl.ds(s0, _CH)])

    prep_fire(0, bufs[0], sems[0])

    def pair_body(ck2, carry):
        ck = ck2 * 2
        prep_fire(ck + 1, bufs[1], sems[1])
        finish(ck, bufs[0], sems[0])

        @pl.when(ck2 * 2 + 2 < nchunk)
        def _():
            prep_fire(ck + 2, bufs[0], sems[0])
        finish(ck + 1, bufs[1], sems[1])
        return carry
    lax.fori_loop(0, nchunk // 2, pair_body, 0)


def _bag(keys, pol_flat, vf_flat):
    B = keys.shape[0]
    mesh = plsc.VectorSubcoreMesh(core_axis_name="c", subcore_axis_name="s")
    f = pl.kernel(
        _bag_body,
        out_type=(jax.ShapeDtypeStruct((B * _NUM_OUT,), jnp.float32),
                  jax.ShapeDtypeStruct((B,), jnp.float32)),
        mesh=mesh,
        compiler_params=pltpu.CompilerParams(needs_layout_passes=False),
        scratch_types=(
            ([pltpu.VMEM((_KPC,), jnp.int32)] * _NUM_OUT +
             [pltpu.VMEM((_KPC,), jnp.float32)] * _NUM_OUT +
             [pltpu.VMEM((_KPC,), jnp.int32),
              pltpu.VMEM((_KPC,), jnp.float32)]
             ) * 2 +
            [
                pltpu.VMEM((_CH * _NUM_OUT,), jnp.float32),
                pltpu.VMEM((_CH,), jnp.float32),
                pltpu.SemaphoreType.DMA,
                pltpu.SemaphoreType.DMA,
            ]),
    )
    keys_t = (keys.reshape(B // _CH, _CH, _N_NODES)
              .transpose(0, 2, 1).reshape(-1))
    return f(keys_t, pol_flat, vf_flat)


def kernel(obs, policy_weight, vf_weight, mapping):
    B = obs.shape[0]
    n_emb = policy_weight.shape[0]
    pol_flat = (policy_weight.reshape(n_emb // 128, 128, _NUM_OUT)
                .transpose(0, 2, 1).reshape(-1))
    vf_flat = vf_weight.reshape(-1)
    h = B // 4
    pol_parts, vf_parts = [], []
    for part in range(4):
        keys = _compute_keys(obs, mapping, part * h, h)
        pol_p, vf_p = _bag(keys, pol_flat, vf_flat)
        pol_parts.append(pol_p)
        vf_parts.append(vf_p)
    pol_out = jnp.concatenate(pol_parts).reshape(B, _NUM_OUT)
    vf_out = jnp.concatenate(vf_parts)
    return pol_out, vf_out


# speedup vs baseline: 1.4456x; 1.1228x over previous
import jax
import jax.numpy as jnp
import numpy as np
from jax import lax
from jax.experimental import pallas as pl
from jax.experimental.pallas import tpu as pltpu
from jax.experimental.pallas import tpu_sc as plsc

_RES = 256
_TUPLE = 16
_OBS_DIM = 8
_INPUT_DIM = _OBS_DIM * _RES
_N_NODES = _INPUT_DIM // _TUPLE
_NUM_OUT = 8
_ENC_MIN, _ENC_MAX = -1.0, 1.0


def _keys_body(obs_ref, onehot_ref, thr_ref, pow_ref, keys_ref):
    vals = jnp.dot(obs_ref[...], onehot_ref[...],
                   precision=lax.Precision.HIGHEST,
                   preferred_element_type=jnp.float32)
    sel = jnp.where(vals > thr_ref[...], pow_ref[...], 0.0)
    acc = sel[:, 0:_N_NODES]
    for t in range(1, _TUPLE):
        acc = acc + sel[:, t * _N_NODES:(t + 1) * _N_NODES]
    offs = lax.broadcasted_iota(jnp.int32, acc.shape, 1) * (2 ** _TUPLE)
    keys_ref[...] = acc.astype(jnp.int32) + offs


def _compute_keys(obs, mapping, row0, rows):
    B = obs.shape[0]
    j2 = np.arange(_INPUT_DIM)
    mapping_t = mapping.reshape(_N_NODES, _TUPLE).T.reshape(-1)
    perm_d = mapping_t // _RES
    perm_t = mapping_t % _RES
    thr_perm = (_ENC_MIN + (_ENC_MAX - _ENC_MIN) *
                (perm_t.astype(jnp.float32) + 1.0) / (_RES + 1.0)
                ).reshape(1, _INPUT_DIM)
    onehot = (perm_d[None, :] == jnp.arange(_OBS_DIM, dtype=jnp.int32)[:, None]
              ).astype(jnp.float32)
    pow_row = jnp.asarray((2.0 ** (j2 // _N_NODES))[None, :],
                          dtype=jnp.float32)

    bt = 512
    base = row0 // bt
    return pl.pallas_call(
        _keys_body,
        grid=(rows // bt,),
        in_specs=[
            pl.BlockSpec((bt, _OBS_DIM), lambda i: (i + base, 0)),
            pl.BlockSpec((_OBS_DIM, _INPUT_DIM), lambda i: (0, 0)),
            pl.BlockSpec((1, _INPUT_DIM), lambda i: (0, 0)),
            pl.BlockSpec((1, _INPUT_DIM), lambda i: (0, 0)),
        ],
        out_specs=pl.BlockSpec((bt, _N_NODES), lambda i: (i, 0)),
        out_shape=jax.ShapeDtypeStruct((rows, _N_NODES), jnp.int32),
    )(obs, onehot, thr_perm, pow_row)


_CH = 16
_NW = 32
_GRP = _N_NODES // 16


_KPC = _CH * _N_NODES


def _bag_body(keys_hbm, pol_hbm, vf_hbm, pol_out, vf_out, *scr):
    nbuf = _NUM_OUT + _NUM_OUT + 2
    bufs = [scr[b * nbuf:(b + 1) * nbuf] for b in range(2)]
    polbuf, vfbuf = scr[2 * nbuf:2 * nbuf + 2]
    sems = scr[2 * nbuf + 2:]
    nc = 2
    wid = lax.axis_index("s") * nc + lax.axis_index("c")
    spw = keys_hbm.shape[0] // (_NW * _N_NODES)
    nchunk = spw // _CH
    lane = lax.iota(jnp.int32, 16)
    zero = jnp.zeros((16,), jnp.float32)

    def prep_fire(ck, buf, sem):
        pidx = buf[:_NUM_OUT]
        gath = buf[_NUM_OUT:2 * _NUM_OUT]
        keys_v, vfg = buf[2 * _NUM_OUT:]
        g0 = (wid * nchunk + ck) * _KPC
        pltpu.sync_copy(keys_hbm.at[pl.ds(g0, _KPC)], keys_v)

        def idx_body(j, c2):
            kk = keys_v[pl.ds(j * 16, 16)]
            b0 = ((kk >> 7) << 10) | (kk & 127)
            for comp in range(_NUM_OUT):
                pidx[comp][pl.ds(j * 16, 16)] = b0 + (comp * 128)
            return c2
        lax.fori_loop(0, _KPC // 16, idx_body, 0)

        for comp in range(_NUM_OUT):
            pltpu.async_copy(pol_hbm.at[pidx[comp]], gath[comp], sem)
        pltpu.async_copy(vf_hbm.at[keys_v], vfg, sem)

    def finish(ck, buf, sem):
        pidx = buf[:_NUM_OUT]
        gath = buf[_NUM_OUT:2 * _NUM_OUT]
        keys_v, vfg = buf[2 * _NUM_OUT:]
        for comp in range(_NUM_OUT):
            pltpu.make_async_copy(pol_hbm.at[pidx[comp]],
                                  gath[comp], sem).wait()
        pltpu.make_async_copy(vf_hbm.at[keys_v], vfg, sem).wait()

        def acc_body(j, accs):
            pol_accs = tuple(
                accs[comp] + gath[comp][pl.ds(j * 16, 16)]
                for comp in range(_NUM_OUT))
            return pol_accs + (accs[_NUM_OUT] + vfg[pl.ds(j * 16, 16)],)
        accs = lax.fori_loop(0, _N_NODES, acc_body, (zero,) * (_NUM_OUT + 1))
        for comp in range(_NUM_OUT):
            plsc.store_scatter(polbuf, [lane * _NUM_OUT + comp], accs[comp])
        vfbuf[...] = accs[_NUM_OUT]

        s0 = wid * spw + ck * _CH
        pltpu.sync_copy(polbuf, pol_out.at[pl.ds(s0 * _NUM_OUT,
                                                 _CH * _NUM_OUT)])
        pltpu.sync_copy(vfbuf, vf_out.at[pl.ds(s0, _CH)])

    prep_fire(0, bufs[0], sems[0])

    def pair_body(ck2, carry):
        ck = ck2 * 2
        prep_fire(ck + 1, bufs[1], sems[1])
        finish(ck, bufs[0], sems[0])

        @pl.when(ck2 * 2 + 2 < nchunk)
        def _():
            prep_fire(ck + 2, bufs[0], sems[0])
        finish(ck + 1, bufs[1], sems[1])
        return carry
    lax.fori_loop(0, nchunk // 2, pair_body, 0)


def _bag(keys, pol_flat, vf_flat):
    B = keys.shape[0]
    mesh = plsc.VectorSubcoreMesh(core_axis_name="c", subcore_axis_name="s")
    f = pl.kernel(
        _bag_body,
        out_type=(jax.ShapeDtypeStruct((B * _NUM_OUT,), jnp.float32),
                  jax.ShapeDtypeStruct((B,), jnp.float32)),
        mesh=mesh,
        compiler_params=pltpu.CompilerParams(needs_layout_passes=False),
        scratch_types=(
            ([pltpu.VMEM((_KPC,), jnp.int32)] * _NUM_OUT +
             [pltpu.VMEM((_KPC,), jnp.float32)] * _NUM_OUT +
             [pltpu.VMEM((_KPC,), jnp.int32),
              pltpu.VMEM((_KPC,), jnp.float32)]
             ) * 2 +
            [
                pltpu.VMEM((_CH * _NUM_OUT,), jnp.float32),
                pltpu.VMEM((_CH,), jnp.float32),
                pltpu.SemaphoreType.DMA,
                pltpu.SemaphoreType.DMA,
            ]),
    )
    keys_t = (keys.reshape(B // _CH, _CH, _N_NODES)
              .transpose(0, 2, 1).reshape(-1))
    return f(keys_t, pol_flat, vf_flat)


def kernel(obs, policy_weight, vf_weight, mapping):
    B = obs.shape[0]
    n_emb = policy_weight.shape[0]
    pol_flat = (policy_weight.reshape(n_emb // 128, 128, _NUM_OUT)
                .transpose(0, 2, 1).reshape(-1))
    vf_flat = vf_weight.reshape(-1)
    h = B // 4
    pol_parts, vf_parts = [], []
    for part in range(4):
        keys = _compute_keys(obs, mapping, part * h, h)
        pol_p, vf_p = _bag(keys, pol_flat, vf_flat)
        pol_parts.append(pol_p)
        vf_parts.append(vf_p)
    pol_out = jnp.concatenate(pol_parts).reshape(B, _NUM_OUT)
    vf_out = jnp.concatenate(vf_parts)
    return pol_out, vf_out


# speedup vs baseline: 1.4496x; 1.0028x over previous
import jax
import jax.numpy as jnp
import numpy as np
from jax import lax
from jax.experimental import pallas as pl
from jax.experimental.pallas import tpu as pltpu
from jax.experimental.pallas import tpu_sc as plsc

_RES = 256
_TUPLE = 16
_OBS_DIM = 8
_INPUT_DIM = _OBS_DIM * _RES
_N_NODES = _INPUT_DIM // _TUPLE
_NUM_OUT = 8
_ENC_MIN, _ENC_MAX = -1.0, 1.0


def _keys_body(obs_ref, onehot_ref, thr_ref, pow_ref, keys_ref):
    vals = jnp.dot(obs_ref[...], onehot_ref[...],
                   precision=lax.Precision.HIGHEST,
                   preferred_element_type=jnp.float32)
    sel = jnp.where(vals > thr_ref[...], pow_ref[...], 0.0)
    acc = sel[:, 0:_N_NODES]
    for t in range(1, _TUPLE):
        acc = acc + sel[:, t * _N_NODES:(t + 1) * _N_NODES]
    offs = lax.broadcasted_iota(jnp.int32, acc.shape, 1) * (2 ** _TUPLE)
    keys_ref[...] = acc.astype(jnp.int32) + offs


def _compute_keys(obs, mapping, row0, rows):
    B = obs.shape[0]
    j2 = np.arange(_INPUT_DIM)
    mapping_t = mapping.reshape(_N_NODES, _TUPLE).T.reshape(-1)
    perm_d = mapping_t // _RES
    perm_t = mapping_t % _RES
    thr_perm = (_ENC_MIN + (_ENC_MAX - _ENC_MIN) *
                (perm_t.astype(jnp.float32) + 1.0) / (_RES + 1.0)
                ).reshape(1, _INPUT_DIM)
    onehot = (perm_d[None, :] == jnp.arange(_OBS_DIM, dtype=jnp.int32)[:, None]
              ).astype(jnp.float32)
    pow_row = jnp.asarray((2.0 ** (j2 // _N_NODES))[None, :],
                          dtype=jnp.float32)

    bt = 512
    base = row0 // bt
    return pl.pallas_call(
        _keys_body,
        grid=(rows // bt,),
        in_specs=[
            pl.BlockSpec((bt, _OBS_DIM), lambda i: (i + base, 0)),
            pl.BlockSpec((_OBS_DIM, _INPUT_DIM), lambda i: (0, 0)),
            pl.BlockSpec((1, _INPUT_DIM), lambda i: (0, 0)),
            pl.BlockSpec((1, _INPUT_DIM), lambda i: (0, 0)),
        ],
        out_specs=pl.BlockSpec((bt, _N_NODES), lambda i: (i, 0)),
        out_shape=jax.ShapeDtypeStruct((rows, _N_NODES), jnp.int32),
    )(obs, onehot, thr_perm, pow_row)


_CH = 16
_NW = 32
_GRP = _N_NODES // 16


_KPC = _CH * _N_NODES


def _bag_body(keys_hbm, pol_hbm, vf_hbm, pol_out, vf_out, *scr):
    nbuf = 1 + _NUM_OUT + 2
    bufs = [scr[b * nbuf:(b + 1) * nbuf] for b in range(2)]
    polbuf, vfbuf = scr[2 * nbuf:2 * nbuf + 2]
    sems = scr[2 * nbuf + 2:]
    nc = 2
    wid = lax.axis_index("s") * nc + lax.axis_index("c")
    spw = keys_hbm.shape[0] // (_NW * _N_NODES)
    nchunk = spw // _CH
    lane = lax.iota(jnp.int32, 16)
    zero = jnp.zeros((16,), jnp.float32)

    npol = pol_hbm.shape[0]

    def prep_fire(ck, buf, sem):
        pidx = buf[0]
        gath = buf[1:1 + _NUM_OUT]
        keys_v, vfg = buf[1 + _NUM_OUT:]
        g0 = (wid * nchunk + ck) * _KPC
        pltpu.sync_copy(keys_hbm.at[pl.ds(g0, _KPC)], keys_v)
        pltpu.async_copy(vf_hbm.at[keys_v], vfg, sem)

        def idx_body(j, c2):
            kk = keys_v[pl.ds(j * 16, 16)]
            pidx[pl.ds(j * 16, 16)] = ((kk >> 7) << 10) | (kk & 127)
            return c2
        lax.fori_loop(0, _KPC // 16, idx_body, 0)

        for comp in range(_NUM_OUT):
            pltpu.async_copy(
                pol_hbm.at[pl.ds(comp * 128, npol - comp * 128)].at[pidx],
                gath[comp], sem)

    def finish(ck, buf, sem):
        pidx = buf[0]
        gath = buf[1:1 + _NUM_OUT]
        keys_v, vfg = buf[1 + _NUM_OUT:]
        for comp in range(_NUM_OUT):
            pltpu.make_async_copy(
                pol_hbm.at[pl.ds(comp * 128, npol - comp * 128)].at[pidx],
                gath[comp], sem).wait()
        pltpu.make_async_copy(vf_hbm.at[keys_v], vfg, sem).wait()

        def acc_body(j, accs):
            pol_accs = tuple(
                accs[comp] + gath[comp][pl.ds(j * 16, 16)]
                for comp in range(_NUM_OUT))
            return pol_accs + (accs[_NUM_OUT] + vfg[pl.ds(j * 16, 16)],)
        accs = lax.fori_loop(0, _N_NODES, acc_body, (zero,) * (_NUM_OUT + 1))
        for comp in range(_NUM_OUT):
            plsc.store_scatter(polbuf, [lane * _NUM_OUT + comp], accs[comp])
        vfbuf[...] = accs[_NUM_OUT]

        s0 = wid * spw + ck * _CH
        pltpu.sync_copy(polbuf, pol_out.at[pl.ds(s0 * _NUM_OUT,
                                                 _CH * _NUM_OUT)])
        pltpu.sync_copy(vfbuf, vf_out.at[pl.ds(s0, _CH)])

    prep_fire(0, bufs[0], sems[0])

    def pair_body(ck2, carry):
        ck = ck2 * 2
        prep_fire(ck + 1, bufs[1], sems[1])
        finish(ck, bufs[0], sems[0])

        @pl.when(ck2 * 2 + 2 < nchunk)
        def _():
            prep_fire(ck + 2, bufs[0], sems[0])
        finish(ck + 1, bufs[1], sems[1])
        return carry
    lax.fori_loop(0, nchunk // 2, pair_body, 0)


def _bag(keys, pol_flat, vf_flat):
    B = keys.shape[0]
    mesh = plsc.VectorSubcoreMesh(core_axis_name="c", subcore_axis_name="s")
    f = pl.kernel(
        _bag_body,
        out_type=(jax.ShapeDtypeStruct((B * _NUM_OUT,), jnp.float32),
                  jax.ShapeDtypeStruct((B,), jnp.float32)),
        mesh=mesh,
        compiler_params=pltpu.CompilerParams(needs_layout_passes=False),
        scratch_types=(
            ([pltpu.VMEM((_KPC,), jnp.int32)] +
             [pltpu.VMEM((_KPC,), jnp.float32)] * _NUM_OUT +
             [pltpu.VMEM((_KPC,), jnp.int32),
              pltpu.VMEM((_KPC,), jnp.float32)]
             ) * 2 +
            [
                pltpu.VMEM((_CH * _NUM_OUT,), jnp.float32),
                pltpu.VMEM((_CH,), jnp.float32),
                pltpu.SemaphoreType.DMA,
                pltpu.SemaphoreType.DMA,
            ]),
    )
    keys_t = (keys.reshape(B // _CH, _CH, _N_NODES)
              .transpose(0, 2, 1).reshape(-1))
    return f(keys_t, pol_flat, vf_flat)


def kernel(obs, policy_weight, vf_weight, mapping):
    B = obs.shape[0]
    n_emb = policy_weight.shape[0]
    pol_flat = (policy_weight.reshape(n_emb // 128, 128, _NUM_OUT)
                .transpose(0, 2, 1).reshape(-1))
    vf_flat = vf_weight.reshape(-1)
    h = B // 4
    pol_parts, vf_parts = [], []
    for part in range(4):
        keys = _compute_keys(obs, mapping, part * h, h)
        pol_p, vf_p = _bag(keys, pol_flat, vf_flat)
        pol_parts.append(pol_p)
        vf_parts.append(vf_p)
    pol_out = jnp.concatenate(pol_parts).reshape(B, _NUM_OUT)
    vf_out = jnp.concatenate(vf_parts)
    return pol_out, vf_out


# speedup vs baseline: 1.4505x; 1.0007x over previous
import jax
import jax.numpy as jnp
import numpy as np
from jax import lax
from jax.experimental import pallas as pl
from jax.experimental.pallas import tpu as pltpu
from jax.experimental.pallas import tpu_sc as plsc

_RES = 256
_TUPLE = 16
_OBS_DIM = 8
_INPUT_DIM = _OBS_DIM * _RES
_N_NODES = _INPUT_DIM // _TUPLE
_NUM_OUT = 8
_ENC_MIN, _ENC_MAX = -1.0, 1.0


def _keys_body(obs_ref, onehot_ref, thr_ref, pow_ref, keys_ref):
    vals = jnp.dot(obs_ref[...], onehot_ref[...],
                   precision=lax.Precision.HIGHEST,
                   preferred_element_type=jnp.float32)
    sel = jnp.where(vals > thr_ref[...], pow_ref[...], 0.0)
    acc = sel[:, 0:_N_NODES]
    for t in range(1, _TUPLE):
        acc = acc + sel[:, t * _N_NODES:(t + 1) * _N_NODES]
    offs = lax.broadcasted_iota(jnp.int32, acc.shape, 1) * (2 ** _TUPLE)
    keys_ref[...] = acc.astype(jnp.int32) + offs


def _compute_keys(obs, mapping, row0, rows):
    B = obs.shape[0]
    j2 = np.arange(_INPUT_DIM)
    mapping_t = mapping.reshape(_N_NODES, _TUPLE).T.reshape(-1)
    perm_d = mapping_t // _RES
    perm_t = mapping_t % _RES
    thr_perm = (_ENC_MIN + (_ENC_MAX - _ENC_MIN) *
                (perm_t.astype(jnp.float32) + 1.0) / (_RES + 1.0)
                ).reshape(1, _INPUT_DIM)
    onehot = (perm_d[None, :] == jnp.arange(_OBS_DIM, dtype=jnp.int32)[:, None]
              ).astype(jnp.float32)
    pow_row = jnp.asarray((2.0 ** (j2 // _N_NODES))[None, :],
                          dtype=jnp.float32)

    bt = 512
    base = row0 // bt
    return pl.pallas_call(
        _keys_body,
        grid=(rows // bt,),
        in_specs=[
            pl.BlockSpec((bt, _OBS_DIM), lambda i: (i + base, 0)),
            pl.BlockSpec((_OBS_DIM, _INPUT_DIM), lambda i: (0, 0)),
            pl.BlockSpec((1, _INPUT_DIM), lambda i: (0, 0)),
            pl.BlockSpec((1, _INPUT_DIM), lambda i: (0, 0)),
        ],
        out_specs=pl.BlockSpec((bt, _N_NODES), lambda i: (i, 0)),
        out_shape=jax.ShapeDtypeStruct((rows, _N_NODES), jnp.int32),
    )(obs, onehot, thr_perm, pow_row)


_CH = 16
_NW = 32


_KPC = _CH * _N_NODES


def _bag_body(keys_hbm, pol_hbm, vf_hbm, pol_out, vf_out, *scr):
    nbuf = 1 + _NUM_OUT + 2
    bufs = [scr[b * nbuf:(b + 1) * nbuf] for b in range(2)]
    polbuf, vfbuf = scr[2 * nbuf:2 * nbuf + 2]
    sems = scr[2 * nbuf + 2:]
    nc = 2
    wid = lax.axis_index("s") * nc + lax.axis_index("c")
    spw = keys_hbm.shape[0] // (_NW * _N_NODES)
    nchunk = spw // _CH
    lane = lax.iota(jnp.int32, 16)
    zero = jnp.zeros((16,), jnp.float32)

    npol = pol_hbm.shape[0]

    def prep_fire(ck, buf, sem):
        pidx = buf[0]
        gath = buf[1:1 + _NUM_OUT]
        keys_v, vfg = buf[1 + _NUM_OUT:]
        g0 = (wid * nchunk + ck) * _KPC
        pltpu.sync_copy(keys_hbm.at[pl.ds(g0, _KPC)], keys_v)
        pltpu.async_copy(vf_hbm.at[keys_v], vfg, sem)

        def idx_body(j, c2):
            kk = keys_v[pl.ds(j * 16, 16)]
            pidx[pl.ds(j * 16, 16)] = ((kk >> 7) << 10) | (kk & 127)
            return c2
        lax.fori_loop(0, _KPC // 16, idx_body, 0)

        for comp in range(_NUM_OUT):
            pltpu.async_copy(
                pol_hbm.at[pl.ds(comp * 128, npol - comp * 128)].at[pidx],
                gath[comp], sem)

    def finish(ck, buf, sem):
        pidx = buf[0]
        gath = buf[1:1 + _NUM_OUT]
        keys_v, vfg = buf[1 + _NUM_OUT:]
        for comp in range(_NUM_OUT):
            pltpu.make_async_copy(
                pol_hbm.at[pl.ds(comp * 128, npol - comp * 128)].at[pidx],
                gath[comp], sem).wait()
        pltpu.make_async_copy(vf_hbm.at[keys_v], vfg, sem).wait()

        def acc_body(j, accs):
            pol_accs = tuple(
                accs[comp] + gath[comp][pl.ds(j * 16, 16)]
                for comp in range(_NUM_OUT))
            return pol_accs + (accs[_NUM_OUT] + vfg[pl.ds(j * 16, 16)],)
        accs = lax.fori_loop(0, _N_NODES, acc_body, (zero,) * (_NUM_OUT + 1))
        for comp in range(_NUM_OUT):
            plsc.store_scatter(polbuf, [lane * _NUM_OUT + comp], accs[comp])
        vfbuf[...] = accs[_NUM_OUT]

        s0 = wid * spw + ck * _CH
        pltpu.sync_copy(polbuf, pol_out.at[pl.ds(s0 * _NUM_OUT,
                                                 _CH * _NUM_OUT)])
        pltpu.sync_copy(vfbuf, vf_out.at[pl.ds(s0, _CH)])

    prep_fire(0, bufs[0], sems[0])

    def pair_body(ck2, carry):
        ck = ck2 * 2
        prep_fire(ck + 1, bufs[1], sems[1])
        finish(ck, bufs[0], sems[0])

        @pl.when(ck2 * 2 + 2 < nchunk)
        def _():
            prep_fire(ck + 2, bufs[0], sems[0])
        finish(ck + 1, bufs[1], sems[1])
        return carry
    lax.fori_loop(0, nchunk // 2, pair_body, 0)


def _bag(keys, pol_flat, vf_flat):
    B = keys.shape[0]
    mesh = plsc.VectorSubcoreMesh(core_axis_name="c", subcore_axis_name="s")
    f = pl.kernel(
        _bag_body,
        out_type=(jax.ShapeDtypeStruct((B * _NUM_OUT,), jnp.float32),
                  jax.ShapeDtypeStruct((B,), jnp.float32)),
        mesh=mesh,
        compiler_params=pltpu.CompilerParams(needs_layout_passes=False),
        scratch_types=(
            ([pltpu.VMEM((_KPC,), jnp.int32)] +
             [pltpu.VMEM((_KPC,), jnp.float32)] * _NUM_OUT +
             [pltpu.VMEM((_KPC,), jnp.int32),
              pltpu.VMEM((_KPC,), jnp.float32)]
             ) * 2 +
            [
                pltpu.VMEM((_CH * _NUM_OUT,), jnp.float32),
                pltpu.VMEM((_CH,), jnp.float32),
                pltpu.SemaphoreType.DMA,
                pltpu.SemaphoreType.DMA,
            ]),
    )
    keys_t = (keys.reshape(B // _CH, _CH, _N_NODES)
              .transpose(0, 2, 1).reshape(-1))
    return f(keys_t, pol_flat, vf_flat)


def kernel(obs, policy_weight, vf_weight, mapping):
    B = obs.shape[0]
    n_emb = policy_weight.shape[0]
    pol_flat = (policy_weight.reshape(n_emb // 128, 128, _NUM_OUT)
                .transpose(0, 2, 1).reshape(-1))
    vf_flat = vf_weight.reshape(-1)
    h = B // 4
    pol_parts, vf_parts = [], []
    for part in range(4):
        keys = _compute_keys(obs, mapping, part * h, h)
        pol_p, vf_p = _bag(keys, pol_flat, vf_flat)
        pol_parts.append(pol_p)
        vf_parts.append(vf_p)
    pol_out = jnp.concatenate(pol_parts).reshape(B, _NUM_OUT)
    vf_out = jnp.concatenate(vf_parts)
    return pol_out, vf_out
